# baseline (speedup 1.0000x reference)
import functools

import jax
import jax.numpy as jnp
from jax import lax
from jax.experimental import pallas as pl
from jax.experimental.pallas import tpu as pltpu
from jax.experimental.pallas import tpu_sc as plsc

N = 10000
E = 160000
D = 256
R = 16
B = 256
H = 128
EPS = 1e-5

NPHASE = 2
DH = D // NPHASE
SPROWS = N + 8
STRIPE = 624
NTILES = 32
EDGES_PER_TILE = E // NTILES
CHUNK = 128
NCHUNKS = 40
PAD = NCHUNKS * CHUNK - EDGES_PER_TILE


def _idx_body(src_ref, et_ref, o_ref):
    o_ref[...] = src_ref[...] * R + et_ref[...]


def _edge_tables(src, etype, dst):
    gidx = pl.pallas_call(
        _idx_body,
        out_shape=jax.ShapeDtypeStruct((E // 128, 128), jnp.int32),
    )(src.reshape(E // 128, 128), etype.reshape(E // 128, 128))
    gidx_t = jnp.pad(gidx.reshape(NTILES, EDGES_PER_TILE), ((0, 0), (0, PAD)))
    dump = jnp.broadcast_to(N + jnp.arange(PAD, dtype=jnp.int32) % 8,
                            (NTILES, PAD))
    dst_t = jnp.concatenate(
        [dst.reshape(NTILES, EDGES_PER_TILE), dump], axis=1)
    return (gidx_t.reshape(NTILES, NCHUNKS, CHUNK),
            dst_t.reshape(NTILES, NCHUNKS, CHUNK))


def _mm_body(x_ref, w_ref, o_ref):
    o_ref[...] = jnp.dot(x_ref[...], w_ref[0],
                         preferred_element_type=jnp.float32)


def _rel_transform_phase(x, w_rel_q):
    rows = 2000
    return pl.pallas_call(
        _mm_body,
        grid=(N // rows, R),
        in_specs=[
            pl.BlockSpec((rows, D), lambda i, j: (i, 0)),
            pl.BlockSpec((1, D, DH), lambda i, j: (j, 0, 0)),
        ],
        out_specs=pl.BlockSpec((rows, DH), lambda i, j: (i, j)),
        out_shape=jax.ShapeDtypeStruct((N, R * DH), jnp.float32),
    )(x, w_rel_q)


def _sc_aggregate_phase(xr_q, gidx_t, dst_t):
    mesh = plsc.VectorSubcoreMesh(core_axis_name="c", subcore_axis_name="s")

    @functools.partial(
        pl.kernel,
        mesh=mesh,
        out_type=jax.ShapeDtypeStruct((2 * N, DH), jnp.float32),
        scratch_types=[
            pltpu.VMEM((NCHUNKS, CHUNK), jnp.int32),
            pltpu.VMEM((NCHUNKS, CHUNK), jnp.int32),
            pltpu.VMEM((2, CHUNK, DH), jnp.float32),
            pltpu.VMEM((16, DH), jnp.float32),
            pltpu.VMEM_SHARED((SPROWS, DH), jnp.float32),
            pltpu.SemaphoreType.DMA,
            pltpu.SemaphoreType.DMA,
        ],
    )
    def k(xr_hbm, gidx_hbm, dst_hbm, out_hbm,
          idx2d, ldst2d, rows2, zero_v, agg_sh, sem0, sem1):
        cid = lax.axis_index("c")
        sid = lax.axis_index("s")
        tid = cid * 16 + sid

        pltpu.sync_copy(gidx_hbm.at[tid], idx2d)
        pltpu.sync_copy(dst_hbm.at[tid], ldst2d)

        nz = DH // 16

        def zfill(i, c):
            zero_v[i // nz, pl.ds((i % nz) * 16, 16)] = jnp.zeros(
                (16,), jnp.float32)
            return c
        lax.fori_loop(0, 16 * nz, zfill, 0)

        base = sid * STRIPE

        def zcopy(i, c):
            pltpu.sync_copy(zero_v, agg_sh.at[pl.ds(base + i * 16, 16)])
            return c
        lax.fori_loop(0, STRIPE // 16, zcopy, 0)

        @pl.when(sid == 15)
        def _():
            pltpu.sync_copy(zero_v, agg_sh.at[pl.ds(16 * STRIPE, 16)])
            pltpu.sync_copy(zero_v.at[pl.ds(0, 8)], agg_sh.at[pl.ds(N, 8)])
        plsc.subcore_barrier()

        pltpu.async_copy(xr_hbm.at[idx2d.at[0]], rows2.at[0], sem0)

        def chunk_body(ch, carry):
            nxt = ch + 1

            @pl.when((nxt < NCHUNKS) & (nxt % 2 == 0))
            def _():
                pltpu.async_copy(xr_hbm.at[idx2d.at[nxt]], rows2.at[0], sem0)

            @pl.when((nxt < NCHUNKS) & (nxt % 2 == 1))
            def _():
                pltpu.async_copy(xr_hbm.at[idx2d.at[nxt]], rows2.at[1], sem1)

            @pl.when(ch % 2 == 0)
            def _():
                pltpu.make_async_copy(xr_hbm.at[pl.ds(0, CHUNK)],
                                      rows2.at[0], sem0).wait()
                pltpu.sync_copy(rows2.at[0], agg_sh.at[ldst2d.at[ch]],
                                add=True)

            @pl.when(ch % 2 == 1)
            def _():
                pltpu.make_async_copy(xr_hbm.at[pl.ds(0, CHUNK)],
                                      rows2.at[1], sem1).wait()
                pltpu.sync_copy(rows2.at[1], agg_sh.at[ldst2d.at[ch]],
                                add=True)
            return carry
        lax.fori_loop(0, NCHUNKS, chunk_body, 0)
        plsc.subcore_barrier()

        obase = cid * N
        pltpu.sync_copy(agg_sh.at[pl.ds(base, STRIPE)],
                        out_hbm.at[pl.ds(obase + base, STRIPE)])

        @pl.when(sid == 15)
        def _():
            pltpu.sync_copy(
                agg_sh.at[pl.ds(16 * STRIPE, N - 16 * STRIPE)],
                out_hbm.at[pl.ds(obase + 16 * STRIPE, N - 16 * STRIPE)])

    return k(xr_q, gidx_t, dst_t)


def _post_body(p0_ref, p1_ref, x_ref, gid_ref,
               b_rel, res_W, res_b, bn_g, bn_b,
               att_w_row, att_b,
               fc1_W, fc1_b, bn1_g, bn1_b,
               fc2_W, fc2_b, bn2_g, bn2_b,
               fc3_W, fc3_b, bn3_g, bn3_b,
               out_W, out_b, o_ref):
    x = x_ref[...]
    p0 = p0_ref[...]
    p1 = p1_ref[...]
    agg = jnp.concatenate([p0[0] + p0[1], p1[0] + p1[1]], axis=1)
    h = jnp.maximum(agg + b_rel[...], 0.0)
    res = jnp.maximum(
        jnp.dot(x, res_W[...], preferred_element_type=jnp.float32)
        + res_b[...], 0.0)
    h = h + res
    m = jnp.mean(h, axis=0, keepdims=True)
    v = jnp.mean((h - m) * (h - m), axis=0, keepdims=True)
    h = (h - m) / jnp.sqrt(v + EPS) * bn_g[...] + bn_b[...]
    z = jnp.sum(h * att_w_row[...], axis=1, keepdims=True) + att_b[...]
    w = 1.0 / (1.0 + jnp.exp(-z))
    hw = h * w
    sel = (lax.broadcasted_iota(jnp.int32, (B, N), 0)
           == gid_ref[...]).astype(jnp.float32)
    g = jnp.dot(sel, hw, preferred_element_type=jnp.float32)

    def fc(t, Wk, bk, gk, btk):
        y = jnp.maximum(
            jnp.dot(t, Wk[...], preferred_element_type=jnp.float32)
            + bk[...], 0.0)
        mm = jnp.mean(y, axis=0, keepdims=True)
        vv = jnp.mean((y - mm) * (y - mm), axis=0, keepdims=True)
        return (y - mm) / jnp.sqrt(vv + EPS) * gk[...] + btk[...]

    h1 = fc(g, fc1_W, fc1_b, bn1_g, bn1_b)
    h2 = fc(h1, fc2_W, fc2_b, bn2_g, bn2_b)
    h3 = fc(h2, fc3_W, fc3_b, bn3_g, bn3_b)
    o_ref[...] = (jnp.dot(h3, out_W[...], preferred_element_type=jnp.float32)
                  + out_b[...])


def _postprocess(p0, p1, x, gid2d, p):
    args = (
        p0, p1, x, gid2d,
        p['b_rel'].reshape(1, D), p['res_W'], p['res_b'].reshape(1, D),
        p['bn_g'].reshape(1, D), p['bn_b'].reshape(1, D),
        p['att_W'].reshape(1, D), p['att_b'].reshape(1, 1),
        p['fc1_W'], p['fc1_b'].reshape(1, H),
        p['bn1_g'].reshape(1, H), p['bn1_b'].reshape(1, H),
        p['fc2_W'], p['fc2_b'].reshape(1, H),
        p['bn2_g'].reshape(1, H), p['bn2_b'].reshape(1, H),
        p['fc3_W'], p['fc3_b'].reshape(1, H),
        p['bn3_g'].reshape(1, H), p['bn3_b'].reshape(1, H),
        p['out_W'], p['out_b'].reshape(1, 1),
    )
    return pl.pallas_call(
        _post_body,
        out_shape=jax.ShapeDtypeStruct((B, 1), jnp.float32),
    )(*args)


def kernel(node_feats, params, edge_index, etype, graph_ids):
    gidx_t, dst_t = _edge_tables(edge_index[0], etype, edge_index[1])
    w = params['W_rel']
    xr0 = _rel_transform_phase(node_feats, w[:, :, :DH])
    part0 = _sc_aggregate_phase(xr0.reshape(N * R, DH), gidx_t, dst_t)
    xr1 = _rel_transform_phase(node_feats, w[:, :, DH:])
    part1 = _sc_aggregate_phase(xr1.reshape(N * R, DH), gidx_t, dst_t)
    gid2d = graph_ids.reshape(1, N)
    return _postprocess(part0.reshape(2, N, DH), part1.reshape(2, N, DH),
                        node_feats, gid2d, params)
